# baseline (speedup 1.0000x reference)
import jax
import jax.numpy as jnp
from jax.experimental import pallas as pl

NUM_HEADS = 12
KEEP_RATIO = 0.7


def _qkv_body(x_ref, w_ref, b_ref, o_ref):
    xb = x_ref[0].astype(jnp.bfloat16)
    wb = w_ref[...].astype(jnp.bfloat16)
    acc = jnp.dot(xb, wb, preferred_element_type=jnp.float32)
    o_ref[0] = acc + b_ref[...]


def _qkv_matmul(x, W_qkv, b_qkv):
    B, N, C = x.shape
    C3 = W_qkv.shape[1]
    return pl.pallas_call(
        _qkv_body,
        grid=(B,),
        in_specs=[
            pl.BlockSpec((1, N, C), lambda b: (b, 0, 0)),
            pl.BlockSpec((C, C3), lambda b: (0, 0)),
            pl.BlockSpec((1, C3), lambda b: (0, 0)),
        ],
        out_specs=pl.BlockSpec((1, N, C3), lambda b: (b, 0, 0)),
        out_shape=jax.ShapeDtypeStruct((B, N, C3), jnp.float32),
    )(x, W_qkv, b_qkv.reshape(1, C3))


def kernel(x, W_qkv, b_qkv, W_proj, b_proj):
    num_heads = NUM_HEADS
    B, N, C = x.shape
    head_dim = C // num_heads
    scale = head_dim ** -0.5
    qkv = _qkv_matmul(x, W_qkv, b_qkv)
    qkv_r0 = qkv.reshape(B, N, 3, num_heads, head_dim).transpose(2, 0, 3, 1, 4)
    q0, k0 = qkv_r0[0], qkv_r0[1]
    attn0 = jax.nn.softmax(jnp.einsum('bhnd,bhmd->bhnm', q0, k0) * scale, axis=-1)
    scores = attn0[:, :, 0, :].mean(axis=1)
    num_patches = N - 1
    keep = max(1, int(KEEP_RATIO * num_patches))
    patch_scores = scores[:, 1:]
    _, idx = jax.lax.top_k(patch_scores, keep)
    idx = jnp.sort(idx, axis=1)
    cls_idx = jnp.zeros((B, 1), dtype=idx.dtype)
    keep_idx = jnp.concatenate([cls_idx, idx + 1], axis=1)
    qkv_g = jnp.take_along_axis(qkv, keep_idx[:, :, None], axis=1)
    Np = qkv_g.shape[1]
    qkv_r = qkv_g.reshape(B, Np, 3, num_heads, head_dim).transpose(2, 0, 3, 1, 4)
    q, k, v = qkv_r[0], qkv_r[1], qkv_r[2]
    attn = jax.nn.softmax(jnp.einsum('bhnd,bhmd->bhnm', q, k) * scale, axis=-1)
    out = jnp.einsum('bhnm,bhmd->bhnd', attn, v).transpose(0, 2, 1, 3).reshape(B, Np, C)
    out = out @ W_proj + b_proj
    next_scores = jnp.take_along_axis(scores, keep_idx, axis=1)
    return (out, keep_idx, next_scores)


# speedup vs baseline: 4.3359x; 4.3359x over previous
import functools

import jax
import jax.numpy as jnp
from jax import lax
from jax.experimental import pallas as pl

_NUM_HEADS = 12
_KEEP_RATIO = 0.7


def _bf(a):
    return a.astype(jnp.bfloat16)


def _fused_body(x_ref, wq_ref, bq_ref, wp_ref, bp_ref,
                out_ref, kidx_ref, nsc_ref, *, N, C, H, keep):
    D = C // H
    NP = keep + 1
    scale = D ** -0.5
    f32 = jnp.float32

    qkv = jnp.dot(_bf(x_ref[0]), _bf(wq_ref[...]),
                  preferred_element_type=f32) + bq_ref[...]

    k_part = qkv[:, C:2 * C]
    q_cls = qkv[0:1, 0:C]
    ic_r = lax.broadcasted_iota(jnp.int32, (C, C), 0)
    ic_c = lax.broadcasted_iota(jnp.int32, (C, C), 1)
    q_col = jnp.sum(jnp.where(ic_r == ic_c, q_cls, 0.0),
                    axis=1, keepdims=True)
    HP = 128
    ih_r = lax.broadcasted_iota(jnp.int32, (C, HP), 0)
    ih_c = lax.broadcasted_iota(jnp.int32, (C, HP), 1)
    m_sel = jnp.where(ih_c == ih_r // D, q_col, 0.0)
    logits = jnp.dot(_bf(k_part), _bf(m_sel),
                     preferred_element_type=f32) * scale
    lmax = jnp.max(logits, axis=0, keepdims=True)
    lexp = jnp.exp(logits - lmax)
    lsum = jnp.sum(lexp, axis=0, keepdims=True)
    probs = lexp / lsum
    head_ok = lax.broadcasted_iota(jnp.int32, (N, HP), 1) < H
    s_col = jnp.sum(jnp.where(head_ok, probs, 0.0),
                    axis=1, keepdims=True) / H

    in_r = lax.broadcasted_iota(jnp.int32, (N, N), 0)
    in_c = lax.broadcasted_iota(jnp.int32, (N, N), 1)
    s_row = jnp.sum(jnp.where(in_r == in_c, s_col, 0.0),
                    axis=0, keepdims=True)
    prefer = ((in_r >= 1) & (in_c >= 1)
              & ((s_col > s_row) | ((s_col == s_row) & (in_r < in_c))))
    rank_row = jnp.sum(prefer.astype(f32), axis=0, keepdims=True)
    kept_row = rank_row < keep
    kept_f = kept_row.astype(f32)
    kept_col = jnp.sum(jnp.where(in_r == in_c, kept_f, 0.0),
                       axis=1, keepdims=True)
    pos_row = jnp.sum(kept_col * (in_r < in_c).astype(f32),
                      axis=0, keepdims=True)

    ip_p = lax.broadcasted_iota(jnp.int32, (NP, N), 0).astype(f32)
    oh = jnp.where(kept_row & (pos_row == ip_p), 1.0, 0.0)

    j_row = lax.broadcasted_iota(jnp.int32, (1, N), 1).astype(f32)
    kidx = jnp.sum(oh * j_row, axis=1, keepdims=True)
    nsc = jnp.sum(oh * s_row, axis=1, keepdims=True)
    kidx_ref[0] = kidx.astype(jnp.int32)
    nsc_ref[0] = nsc

    gath = jnp.dot(_bf(oh), _bf(qkv), preferred_element_type=f32)
    gb = _bf(gath)

    outs = []
    for h in range(H):
        qh = gb[:, h * D:(h + 1) * D]
        kh = gb[:, C + h * D:C + (h + 1) * D]
        vh = gb[:, 2 * C + h * D:2 * C + (h + 1) * D]
        s_att = lax.dot_general(qh, kh, (((1,), (1,)), ((), ())),
                                preferred_element_type=f32) * scale
        amax = jnp.max(s_att, axis=1, keepdims=True)
        aexp = jnp.exp(s_att - amax)
        p_att = aexp / jnp.sum(aexp, axis=1, keepdims=True)
        outs.append(jnp.dot(_bf(p_att), vh, preferred_element_type=f32))
    att = jnp.concatenate(outs, axis=1)
    out_ref[0] = jnp.dot(_bf(att), _bf(wp_ref[...]),
                         preferred_element_type=f32) + bp_ref[...]


def kernel(x, W_qkv, b_qkv, W_proj, b_proj):
    B, N, C = x.shape
    C3 = W_qkv.shape[1]
    H = _NUM_HEADS
    keep = max(1, int(_KEEP_RATIO * (N - 1)))
    NP = keep + 1

    body = functools.partial(_fused_body, N=N, C=C, H=H, keep=keep)
    out, kidx, nsc = pl.pallas_call(
        body,
        grid=(B,),
        in_specs=[
            pl.BlockSpec((1, N, C), lambda b: (b, 0, 0)),
            pl.BlockSpec((C, C3), lambda b: (0, 0)),
            pl.BlockSpec((1, C3), lambda b: (0, 0)),
            pl.BlockSpec((C, C), lambda b: (0, 0)),
            pl.BlockSpec((1, C), lambda b: (0, 0)),
        ],
        out_specs=[
            pl.BlockSpec((1, NP, C), lambda b: (b, 0, 0)),
            pl.BlockSpec((1, NP, 1), lambda b: (b, 0, 0)),
            pl.BlockSpec((1, NP, 1), lambda b: (b, 0, 0)),
        ],
        out_shape=[
            jax.ShapeDtypeStruct((B, NP, C), jnp.float32),
            jax.ShapeDtypeStruct((B, NP, 1), jnp.int32),
            jax.ShapeDtypeStruct((B, NP, 1), jnp.float32),
        ],
    )(x, W_qkv, b_qkv.reshape(1, C3), W_proj, b_proj.reshape(1, C))
    return (out, kidx[..., 0], nsc[..., 0])


# speedup vs baseline: 4.7076x; 1.0857x over previous
import functools

import jax
import jax.numpy as jnp
from jax import lax
from jax.experimental import pallas as pl

_NUM_HEADS = 12
_KEEP_RATIO = 0.7


def _bf(a):
    return a.astype(jnp.bfloat16)


def _fused_body(x_ref, wq_ref, bq_ref, wp_ref, bp_ref,
                out_ref, kidx_ref, nsc_ref, *, N, C, H, keep):
    D = C // H
    NP = keep + 1
    scale = D ** -0.5
    f32 = jnp.float32

    qkv = jnp.dot(x_ref[0], wq_ref[...],
                  preferred_element_type=f32) + bq_ref[...]
    qkvb = _bf(qkv)

    k_part = qkvb[:, C:2 * C]
    q_cls = qkvb[0:1, 0:C].astype(f32) * scale
    ic_r = lax.broadcasted_iota(jnp.int32, (C, C), 0)
    ic_c = lax.broadcasted_iota(jnp.int32, (C, C), 1)
    q_col = jnp.sum(jnp.where(ic_r == ic_c, q_cls, 0.0),
                    axis=1, keepdims=True)
    HP = 128
    ih_r = lax.broadcasted_iota(jnp.int32, (C, HP), 0)
    ih_c = lax.broadcasted_iota(jnp.int32, (C, HP), 1)
    m_sel = jnp.where(ih_c == ih_r // D, q_col, 0.0)
    logits = jnp.dot(k_part, _bf(m_sel),
                     preferred_element_type=f32)
    lmax = jnp.max(logits, axis=0, keepdims=True)
    lexp = jnp.exp(logits - lmax)
    lsum = jnp.sum(lexp, axis=0, keepdims=True)
    probs = lexp / lsum
    head_ok = lax.broadcasted_iota(jnp.int32, (N, HP), 1) < H
    s_col = jnp.sum(jnp.where(head_ok, probs, 0.0),
                    axis=1, keepdims=True) / H

    in_r = lax.broadcasted_iota(jnp.int32, (N, N), 0)
    in_c = lax.broadcasted_iota(jnp.int32, (N, N), 1)
    s_row = jnp.sum(jnp.where(in_r == in_c, s_col, 0.0),
                    axis=0, keepdims=True)
    prefer = ((in_r >= 1) & (in_c >= 1)
              & ((s_col > s_row) | ((s_col == s_row) & (in_r < in_c))))
    rank_row = jnp.sum(prefer.astype(f32), axis=0, keepdims=True)
    kept_row = rank_row < keep
    kept_f = kept_row.astype(f32)
    kept_col = jnp.sum(jnp.where(in_r == in_c, kept_f, 0.0),
                       axis=1, keepdims=True)
    pos_row = jnp.sum(kept_col * (in_r < in_c).astype(f32),
                      axis=0, keepdims=True)

    ip_p = lax.broadcasted_iota(jnp.int32, (NP, N), 0).astype(f32)
    oh = jnp.where(kept_row & (pos_row == ip_p), 1.0, 0.0)

    j_row = lax.broadcasted_iota(jnp.int32, (1, N), 1).astype(f32)
    kidx = jnp.sum(oh * j_row, axis=1, keepdims=True)
    nsc = jnp.sum(oh * s_row, axis=1, keepdims=True)
    kidx_ref[0] = kidx.astype(jnp.int32)
    nsc_ref[0] = nsc

    gb = _bf(jnp.dot(_bf(oh), qkvb,
                     preferred_element_type=f32))

    outs = []
    for h in range(H):
        qh = gb[:, h * D:(h + 1) * D] * jnp.bfloat16(scale)
        kh = gb[:, C + h * D:C + (h + 1) * D]
        vh = gb[:, 2 * C + h * D:2 * C + (h + 1) * D]
        s_att = lax.dot_general(qh, kh, (((1,), (1,)), ((), ())),
                                preferred_element_type=f32)
        aexp = jnp.exp(s_att)
        rs = 1.0 / jnp.sum(aexp, axis=1, keepdims=True)
        o_h = jnp.dot(_bf(aexp), vh, preferred_element_type=f32)
        outs.append(o_h * rs)
    att = jnp.concatenate(outs, axis=1)
    out_ref[0] = jnp.dot(_bf(att), wp_ref[...],
                         preferred_element_type=f32) + bp_ref[...]


def kernel(x, W_qkv, b_qkv, W_proj, b_proj):
    B, N, C = x.shape
    C3 = W_qkv.shape[1]
    H = _NUM_HEADS
    keep = max(1, int(_KEEP_RATIO * (N - 1)))
    NP = keep + 1

    body = functools.partial(_fused_body, N=N, C=C, H=H, keep=keep)
    out, kidx, nsc = pl.pallas_call(
        body,
        grid=(B,),
        in_specs=[
            pl.BlockSpec((1, N, C), lambda b: (b, 0, 0)),
            pl.BlockSpec((C, C3), lambda b: (0, 0)),
            pl.BlockSpec((1, C3), lambda b: (0, 0)),
            pl.BlockSpec((C, C), lambda b: (0, 0)),
            pl.BlockSpec((1, C), lambda b: (0, 0)),
        ],
        out_specs=[
            pl.BlockSpec((1, NP, C), lambda b: (b, 0, 0)),
            pl.BlockSpec((1, NP, 1), lambda b: (b, 0, 0)),
            pl.BlockSpec((1, NP, 1), lambda b: (b, 0, 0)),
        ],
        out_shape=[
            jax.ShapeDtypeStruct((B, NP, C), jnp.float32),
            jax.ShapeDtypeStruct((B, NP, 1), jnp.int32),
            jax.ShapeDtypeStruct((B, NP, 1), jnp.float32),
        ],
    )(_bf(x), _bf(W_qkv), b_qkv.reshape(1, C3), _bf(W_proj),
      b_proj.reshape(1, C))
    return (out, kidx[..., 0], nsc[..., 0])


# speedup vs baseline: 4.7097x; 1.0005x over previous
import functools

import jax
import jax.numpy as jnp
from jax import lax
from jax.experimental import pallas as pl

_NUM_HEADS = 12
_KEEP_RATIO = 0.7


def _bf(a):
    return a.astype(jnp.bfloat16)


def _fused_body(x_ref, wq_ref, bq_ref, wp_ref, bp_ref,
                out_ref, kidx_ref, nsc_ref, *, N, C, H, keep):
    D = C // H
    NP = keep + 1
    scale = D ** -0.5
    f32 = jnp.float32

    qkv = jnp.dot(x_ref[0], wq_ref[...],
                  preferred_element_type=f32) + bq_ref[...]
    qkvb = _bf(qkv)

    k_part = qkvb[:, C:2 * C]
    q_cls = qkvb[0:1, 0:C].astype(f32) * scale
    ic_r = lax.broadcasted_iota(jnp.int32, (C, C), 0)
    ic_c = lax.broadcasted_iota(jnp.int32, (C, C), 1)
    q_col = jnp.sum(jnp.where(ic_r == ic_c, q_cls, 0.0),
                    axis=1, keepdims=True)
    HP = 128
    ih_r = lax.broadcasted_iota(jnp.int32, (C, HP), 0)
    ih_c = lax.broadcasted_iota(jnp.int32, (C, HP), 1)
    m_sel = jnp.where(ih_c == ih_r // D, q_col, 0.0)
    logits = jnp.dot(k_part, _bf(m_sel),
                     preferred_element_type=f32)
    lmax = jnp.max(logits, axis=0, keepdims=True)
    lexp = jnp.exp(logits - lmax)
    lsum = jnp.sum(lexp, axis=0, keepdims=True)
    probs = lexp / lsum
    head_ok = lax.broadcasted_iota(jnp.int32, (N, HP), 1) < H
    s_col = jnp.sum(jnp.where(head_ok, probs, 0.0),
                    axis=1, keepdims=True) / H

    in_r = lax.broadcasted_iota(jnp.int32, (N, N), 0)
    in_c = lax.broadcasted_iota(jnp.int32, (N, N), 1)
    s_row = jnp.sum(jnp.where(in_r == in_c, s_col, 0.0),
                    axis=0, keepdims=True)
    prefer = ((in_r >= 1) & (in_c >= 1)
              & ((s_col > s_row) | ((s_col == s_row) & (in_r < in_c))))
    rank_row = jnp.sum(prefer.astype(f32), axis=0, keepdims=True)
    kept_row = rank_row < keep
    kept_f = kept_row.astype(f32)
    kept_col = jnp.sum(jnp.where(in_r == in_c, kept_f, 0.0),
                       axis=1, keepdims=True)
    pos_row = jnp.sum(kept_col * (in_r < in_c).astype(f32),
                      axis=0, keepdims=True)

    ip_p = lax.broadcasted_iota(jnp.int32, (NP, N), 0).astype(f32)
    ohm = kept_row & (pos_row == ip_p)
    oh = jnp.where(ohm, 1.0, 0.0)

    j_row = lax.broadcasted_iota(jnp.int32, (1, N), 1).astype(f32)
    kidx = jnp.sum(oh * j_row, axis=1, keepdims=True)
    nsc = jnp.sum(oh * s_row, axis=1, keepdims=True)
    kidx_ref[0] = kidx.astype(jnp.int32)
    nsc_ref[0] = nsc

    gb = _bf(jnp.dot(_bf(oh), qkvb,
                     preferred_element_type=f32))

    onescol = (lax.broadcasted_iota(jnp.int32, (NP, D), 1) == 0)
    onescol = onescol.astype(jnp.bfloat16)
    outs = []
    for h in range(H):
        qh = gb[:, h * D:(h + 1) * D] * jnp.bfloat16(scale)
        kh = gb[:, C + h * D:C + (h + 1) * D]
        vh = gb[:, 2 * C + h * D:2 * C + (h + 1) * D]
        s_att = lax.dot_general(qh, kh, (((1,), (1,)), ((), ())),
                                preferred_element_type=f32)
        pb = _bf(jnp.exp(s_att))
        vaug = jnp.concatenate([vh, onescol], axis=1)
        o_aug = jnp.dot(pb, vaug, preferred_element_type=f32)
        rs = 1.0 / o_aug[:, D:D + 1]
        outs.append(o_aug[:, :D] * rs)
    att = jnp.concatenate(outs, axis=1)
    out_ref[0] = jnp.dot(_bf(att), wp_ref[...],
                         preferred_element_type=f32) + bp_ref[...]


def kernel(x, W_qkv, b_qkv, W_proj, b_proj):
    B, N, C = x.shape
    C3 = W_qkv.shape[1]
    H = _NUM_HEADS
    keep = max(1, int(_KEEP_RATIO * (N - 1)))
    NP = keep + 1

    body = functools.partial(_fused_body, N=N, C=C, H=H, keep=keep)
    out, kidx, nsc = pl.pallas_call(
        body,
        grid=(B,),
        in_specs=[
            pl.BlockSpec((1, N, C), lambda b: (b, 0, 0)),
            pl.BlockSpec((C, C3), lambda b: (0, 0)),
            pl.BlockSpec((1, C3), lambda b: (0, 0)),
            pl.BlockSpec((C, C), lambda b: (0, 0)),
            pl.BlockSpec((1, C), lambda b: (0, 0)),
        ],
        out_specs=[
            pl.BlockSpec((1, NP, C), lambda b: (b, 0, 0)),
            pl.BlockSpec((1, NP, 1), lambda b: (b, 0, 0)),
            pl.BlockSpec((1, NP, 1), lambda b: (b, 0, 0)),
        ],
        out_shape=[
            jax.ShapeDtypeStruct((B, NP, C), jnp.float32),
            jax.ShapeDtypeStruct((B, NP, 1), jnp.int32),
            jax.ShapeDtypeStruct((B, NP, 1), jnp.float32),
        ],
    )(_bf(x), _bf(W_qkv), b_qkv.reshape(1, C3), _bf(W_proj),
      b_proj.reshape(1, C))
    return (out, kidx[..., 0], nsc[..., 0])


# speedup vs baseline: 4.7251x; 1.0033x over previous
import functools

import jax
import jax.numpy as jnp
from jax import lax
from jax.experimental import pallas as pl

_NUM_HEADS = 12
_KEEP_RATIO = 0.7


def _bf(a):
    return a.astype(jnp.bfloat16)


def _fused_body(x_ref, wq_ref, bq_ref, wp_ref, bp_ref,
                out_ref, kidx_ref, nsc_ref, *, N, C, H, keep, BB):
    for bi in range(BB):
        _one_batch(x_ref, wq_ref, bq_ref, wp_ref, bp_ref,
                   out_ref, kidx_ref, nsc_ref, bi, N=N, C=C, H=H, keep=keep)


def _one_batch(x_ref, wq_ref, bq_ref, wp_ref, bp_ref,
               out_ref, kidx_ref, nsc_ref, bi, *, N, C, H, keep):
    D = C // H
    NP = keep + 1
    scale = D ** -0.5
    f32 = jnp.float32

    qkv = jnp.dot(x_ref[bi], wq_ref[...],
                  preferred_element_type=f32) + bq_ref[...]
    qkvb = _bf(qkv)

    k_part = qkvb[:, C:2 * C]
    q_cls = qkvb[0:1, 0:C].astype(f32) * scale
    ic_r = lax.broadcasted_iota(jnp.int32, (C, C), 0)
    ic_c = lax.broadcasted_iota(jnp.int32, (C, C), 1)
    q_col = jnp.sum(jnp.where(ic_r == ic_c, q_cls, 0.0),
                    axis=1, keepdims=True)
    HP = 128
    ih_r = lax.broadcasted_iota(jnp.int32, (C, HP), 0)
    ih_c = lax.broadcasted_iota(jnp.int32, (C, HP), 1)
    m_sel = jnp.where(ih_c == ih_r // D, q_col, 0.0)
    logits = jnp.dot(k_part, _bf(m_sel),
                     preferred_element_type=f32)
    lmax = jnp.max(logits, axis=0, keepdims=True)
    lexp = jnp.exp(logits - lmax)
    lsum = jnp.sum(lexp, axis=0, keepdims=True)
    probs = lexp / lsum
    head_ok = lax.broadcasted_iota(jnp.int32, (N, HP), 1) < H
    s_col = jnp.sum(jnp.where(head_ok, probs, 0.0),
                    axis=1, keepdims=True) / H

    in_r = lax.broadcasted_iota(jnp.int32, (N, N), 0)
    in_c = lax.broadcasted_iota(jnp.int32, (N, N), 1)
    s_row = jnp.sum(jnp.where(in_r == in_c, s_col, 0.0),
                    axis=0, keepdims=True)
    prefer = ((in_r >= 1) & (in_c >= 1)
              & ((s_col > s_row) | ((s_col == s_row) & (in_r < in_c))))
    rank_row = jnp.sum(prefer.astype(f32), axis=0, keepdims=True)
    kept_row = rank_row < keep
    kept_f = kept_row.astype(f32)
    kept_col = jnp.sum(jnp.where(in_r == in_c, kept_f, 0.0),
                       axis=1, keepdims=True)
    pos_row = jnp.sum(kept_col * (in_r < in_c).astype(f32),
                      axis=0, keepdims=True)

    ip_p = lax.broadcasted_iota(jnp.int32, (NP, N), 0).astype(f32)
    ohm = kept_row & (pos_row == ip_p)
    oh = jnp.where(ohm, 1.0, 0.0)

    j_row = lax.broadcasted_iota(jnp.int32, (1, N), 1).astype(f32)
    kidx = jnp.sum(oh * j_row, axis=1, keepdims=True)
    nsc = jnp.sum(oh * s_row, axis=1, keepdims=True)
    kidx_ref[bi] = kidx.astype(jnp.int32)
    nsc_ref[bi] = nsc

    gb = _bf(jnp.dot(_bf(oh), qkvb,
                     preferred_element_type=f32))

    onescol = (lax.broadcasted_iota(jnp.int32, (NP, D), 1) == 0)
    onescol = onescol.astype(jnp.bfloat16)
    outs = []
    for h in range(H):
        qh = gb[:, h * D:(h + 1) * D] * jnp.bfloat16(scale)
        kh = gb[:, C + h * D:C + (h + 1) * D]
        vh = gb[:, 2 * C + h * D:2 * C + (h + 1) * D]
        s_att = lax.dot_general(qh, kh, (((1,), (1,)), ((), ())),
                                preferred_element_type=f32)
        pb = _bf(jnp.exp(s_att))
        vaug = jnp.concatenate([vh, onescol], axis=1)
        o_aug = jnp.dot(pb, vaug, preferred_element_type=f32)
        rs = 1.0 / o_aug[:, D:D + 1]
        outs.append(o_aug[:, :D] * rs)
    att = jnp.concatenate(outs, axis=1)
    out_ref[bi] = jnp.dot(_bf(att), wp_ref[...],
                         preferred_element_type=f32) + bp_ref[...]


def kernel(x, W_qkv, b_qkv, W_proj, b_proj):
    B, N, C = x.shape
    C3 = W_qkv.shape[1]
    H = _NUM_HEADS
    keep = max(1, int(_KEEP_RATIO * (N - 1)))
    NP = keep + 1

    BB = 2
    body = functools.partial(_fused_body, N=N, C=C, H=H, keep=keep, BB=BB)
    out, kidx, nsc = pl.pallas_call(
        body,
        grid=(B // BB,),
        in_specs=[
            pl.BlockSpec((BB, N, C), lambda b: (b, 0, 0)),
            pl.BlockSpec((C, C3), lambda b: (0, 0)),
            pl.BlockSpec((1, C3), lambda b: (0, 0)),
            pl.BlockSpec((C, C), lambda b: (0, 0)),
            pl.BlockSpec((1, C), lambda b: (0, 0)),
        ],
        out_specs=[
            pl.BlockSpec((BB, NP, C), lambda b: (b, 0, 0)),
            pl.BlockSpec((BB, NP, 1), lambda b: (b, 0, 0)),
            pl.BlockSpec((BB, NP, 1), lambda b: (b, 0, 0)),
        ],
        out_shape=[
            jax.ShapeDtypeStruct((B, NP, C), jnp.float32),
            jax.ShapeDtypeStruct((B, NP, 1), jnp.int32),
            jax.ShapeDtypeStruct((B, NP, 1), jnp.float32),
        ],
    )(_bf(x), _bf(W_qkv), b_qkv.reshape(1, C3), _bf(W_proj),
      b_proj.reshape(1, C))
    return (out, kidx[..., 0], nsc[..., 0])
